# speedup vs baseline: 9.5315x; 9.5315x over previous
import functools

import jax
import jax.numpy as jnp
from jax.experimental import pallas as pl

N = 16384
S = 4096
QB = 256
MB = 512
D1 = 128
D2 = 256
C0 = 256
C1 = 256


def _knn_body(x1_ref, p1_ref, x2_ref, p2_ref, pts2_ref, pts1_ref, w0_ref,
              b0_ref, y0_ref, stats_ref):
    x1 = x1_ref[...]
    x2 = x2_ref[...]
    q2 = jnp.sum(x1 * x1, axis=0)
    k2 = jnp.sum(x2 * x2, axis=0)
    cross = jax.lax.dot_general(x2, x1, (((0,), (0,)), ((), ())),
                                preferred_element_type=jnp.float32)
    d2 = (k2[:, None] + q2[None, :]) - 2.0 * cross

    p1 = p1_ref[0, :]
    p2 = p2_ref[0, :]
    d = jnp.where(p2[:, None] != p1[None, :], jnp.float32(1e10), d2)

    rows = jax.lax.broadcasted_iota(jnp.int32, (S, QB), 0)
    mins, sels = [], []
    for _ in range(3):
        mk = jnp.min(d, axis=0, keepdims=True)
        ik = jnp.min(jnp.where(d == mk, rows, S), axis=0, keepdims=True)
        sel = rows == ik
        mins.append(mk)
        sels.append(sel)
        d = jnp.where(sel, jnp.float32(jnp.inf), d)

    r = [1.0 / (m + 1e-8) for m in mins]
    norm = r[0] + r[1] + r[2]
    wm = (jnp.where(sels[0], r[0] / norm, 0.0)
          + jnp.where(sels[1], r[1] / norm, 0.0)
          + jnp.where(sels[2], r[2] / norm, 0.0))

    interp = jax.lax.dot_general(pts2_ref[...], wm, (((1,), (0,)), ((), ())),
                                 preferred_element_type=jnp.float32)

    w0 = w0_ref[...]
    y0 = (jax.lax.dot_general(w0[:, :D1], pts1_ref[...],
                              (((1,), (0,)), ((), ())),
                              preferred_element_type=jnp.float32)
          + jax.lax.dot_general(w0[:, D1:], interp,
                                (((1,), (0,)), ((), ())),
                                preferred_element_type=jnp.float32)
          + b0_ref[...])
    y0_ref[...] = y0

    s1 = jnp.sum(y0, axis=1)
    s2 = jnp.sum(y0 * y0, axis=1)
    blk = jnp.concatenate(
        [s1[None, :], s2[None, :], jnp.zeros((6, C0), jnp.float32)], axis=0)

    @pl.when(pl.program_id(0) == 0)
    def _():
        stats_ref[...] = blk

    @pl.when(pl.program_id(0) != 0)
    def _():
        stats_ref[...] += blk


def _mid_body(y0_ref, sc_ref, sh_ref, w1_ref, b1_ref, y1_ref, stats_ref):
    h = jnp.maximum(y0_ref[...] * sc_ref[...] + sh_ref[...], 0.0)
    y1 = jax.lax.dot_general(w1_ref[...], h, (((1,), (0,)), ((), ())),
                             preferred_element_type=jnp.float32) + b1_ref[...]
    y1_ref[...] = y1
    s1 = jnp.sum(y1, axis=1)
    s2 = jnp.sum(y1 * y1, axis=1)
    blk = jnp.concatenate(
        [s1[None, :], s2[None, :], jnp.zeros((6, C1), jnp.float32)], axis=0)

    @pl.when(pl.program_id(0) == 0)
    def _():
        stats_ref[...] = blk

    @pl.when(pl.program_id(0) != 0)
    def _():
        stats_ref[...] += blk


def _final_body(y1_ref, sc_ref, sh_ref, out_ref):
    out_ref[...] = jnp.maximum(y1_ref[...] * sc_ref[...] + sh_ref[...], 0.0)


def _affine(stats, g, beta, n):
    mean = stats[0] / n
    var = stats[1] / n - mean * mean
    scale = g / jnp.sqrt(var + 1e-5)
    shift = beta - mean * scale
    return scale.reshape(-1, 1), shift.reshape(-1, 1)


@jax.jit
def kernel(xyz1, xyz2, piece_id1, piece_id2, points1, points2,
           conv_w0, conv_b0, bn_g0, bn_b0, conv_w1, conv_b1, bn_g1, bn_b1):
    x1 = xyz1[0]
    x2 = xyz2[0]
    p1 = piece_id1.reshape(1, N)
    p2 = piece_id2.reshape(1, S)
    pts1 = points1[0]
    pts2 = points2[0]
    b0 = conv_b0.reshape(C0, 1)
    b1 = conv_b1.reshape(C1, 1)

    grid1 = N // QB
    y0, stats0 = pl.pallas_call(
        _knn_body,
        grid=(grid1,),
        in_specs=[
            pl.BlockSpec((3, QB), lambda i: (0, i)),
            pl.BlockSpec((1, QB), lambda i: (0, i)),
            pl.BlockSpec((3, S), lambda i: (0, 0)),
            pl.BlockSpec((1, S), lambda i: (0, 0)),
            pl.BlockSpec((D2, S), lambda i: (0, 0)),
            pl.BlockSpec((D1, QB), lambda i: (0, i)),
            pl.BlockSpec((C0, D1 + D2), lambda i: (0, 0)),
            pl.BlockSpec((C0, 1), lambda i: (0, 0)),
        ],
        out_specs=[
            pl.BlockSpec((C0, QB), lambda i: (0, i)),
            pl.BlockSpec((8, C0), lambda i: (0, 0)),
        ],
        out_shape=[
            jax.ShapeDtypeStruct((C0, N), jnp.float32),
            jax.ShapeDtypeStruct((8, C0), jnp.float32),
        ],
    )(x1, p1, x2, p2, pts2, pts1, conv_w0, b0)

    sc0, sh0 = _affine(stats0, bn_g0, bn_b0, N)

    grid2 = N // MB
    y1, stats1 = pl.pallas_call(
        _mid_body,
        grid=(grid2,),
        in_specs=[
            pl.BlockSpec((C0, MB), lambda i: (0, i)),
            pl.BlockSpec((C0, 1), lambda i: (0, 0)),
            pl.BlockSpec((C0, 1), lambda i: (0, 0)),
            pl.BlockSpec((C1, C0), lambda i: (0, 0)),
            pl.BlockSpec((C1, 1), lambda i: (0, 0)),
        ],
        out_specs=[
            pl.BlockSpec((C1, MB), lambda i: (0, i)),
            pl.BlockSpec((8, C1), lambda i: (0, 0)),
        ],
        out_shape=[
            jax.ShapeDtypeStruct((C1, N), jnp.float32),
            jax.ShapeDtypeStruct((8, C1), jnp.float32),
        ],
    )(y0, sc0, sh0, conv_w1, b1)

    sc1, sh1 = _affine(stats1, bn_g1, bn_b1, N)

    out = pl.pallas_call(
        _final_body,
        grid=(grid2,),
        in_specs=[
            pl.BlockSpec((C1, MB), lambda i: (0, i)),
            pl.BlockSpec((C1, 1), lambda i: (0, 0)),
            pl.BlockSpec((C1, 1), lambda i: (0, 0)),
        ],
        out_specs=pl.BlockSpec((C1, MB), lambda i: (0, i)),
        out_shape=jax.ShapeDtypeStruct((C1, N), jnp.float32),
    )(y1, sc1, sh1)

    return out[None]


# speedup vs baseline: 18.3356x; 1.9237x over previous
import jax
import jax.numpy as jnp
from jax.experimental import pallas as pl
from jax.experimental.pallas import tpu as pltpu

N = 16384
S = 4096
QB = 256
CB = 512
NC = S // CB
MB = 512
D1 = 128
D2 = 256
C0 = 256
C1 = 256
MASKED = 1e10


def _merge_sorted3(a, b):
    a1, a2, a3 = a
    b1, b2, b3 = b
    r1 = jnp.minimum(a1, b1)
    lo2 = jnp.minimum(a2, b2)
    t = jnp.maximum(a1, b1)
    r2 = jnp.minimum(t, lo2)
    r3 = jnp.minimum(jnp.maximum(t, lo2),
                     jnp.minimum(jnp.maximum(a2, b2), jnp.minimum(a3, b3)))
    return r1, r2, r3


def _knn_body(info_ref, x1_ref, p1_ref, x2_ref, p2_ref, pts2_ref, pts1_ref,
              w0_ref, b0_ref, y0_ref, stats_ref, dscr, acc, mins):
    i = pl.program_id(0)
    pmin = info_ref[2 * NC + 2 * i]
    pmax = info_ref[2 * NC + 2 * i + 1]

    x1 = x1_ref[...]
    q2 = jnp.sum(x1 * x1, axis=0)
    p1 = p1_ref[0, :]

    mins[...] = jnp.full((8, QB), jnp.inf, jnp.float32)
    acc[...] = jnp.zeros((D2, QB), jnp.float32)

    for c in range(NC):
        active = (info_ref[2 * c + 1] >= pmin) & (info_ref[2 * c] <= pmax)

        @pl.when(active)
        def _(c=c):
            x2c = x2_ref[:, c * CB:(c + 1) * CB]
            k2c = jnp.sum(x2c * x2c, axis=0)
            crossc = jax.lax.dot_general(
                x2c, x1, (((0,), (0,)), ((), ())),
                preferred_element_type=jnp.float32)
            d2c = (k2c[:, None] + q2[None, :]) - 2.0 * crossc
            p2c = p2_ref[0, c * CB:(c + 1) * CB]
            dm = jnp.where(p2c[:, None] != p1[None, :], MASKED, d2c)
            dscr[c * CB:(c + 1) * CB, :] = dm
            c1 = jnp.min(dm, axis=0, keepdims=True)
            dm2 = jnp.where(dm == c1, jnp.inf, dm)
            c2 = jnp.min(dm2, axis=0, keepdims=True)
            dm3 = jnp.where(dm2 == c2, jnp.inf, dm2)
            c3 = jnp.min(dm3, axis=0, keepdims=True)
            m1, m2, m3 = _merge_sorted3(
                (mins[0:1, :], mins[1:2, :], mins[2:3, :]), (c1, c2, c3))
            mins[0:1, :] = m1
            mins[1:2, :] = m2
            mins[2:3, :] = m3

    m1, m2, m3 = mins[0:1, :], mins[1:2, :], mins[2:3, :]
    r1 = 1.0 / (m1 + 1e-8)
    r2 = 1.0 / (m2 + 1e-8)
    r3 = 1.0 / (m3 + 1e-8)
    norm = r1 + r2 + r3
    w1 = r1 / norm
    w2 = r2 / norm
    w3 = r3 / norm

    for c in range(NC):
        active = (info_ref[2 * c + 1] >= pmin) & (info_ref[2 * c] <= pmax)

        @pl.when(active)
        def _(c=c):
            dm = dscr[c * CB:(c + 1) * CB, :]
            wc = (jnp.where(dm == m1, w1, 0.0)
                  + jnp.where(dm == m2, w2, 0.0)
                  + jnp.where(dm == m3, w3, 0.0))
            acc[...] += jax.lax.dot_general(
                pts2_ref[:, c * CB:(c + 1) * CB], wc,
                (((1,), (0,)), ((), ())),
                preferred_element_type=jnp.float32)

    w0 = w0_ref[...]
    y0 = (jax.lax.dot_general(w0[:, :D1], pts1_ref[...],
                              (((1,), (0,)), ((), ())),
                              preferred_element_type=jnp.float32)
          + jax.lax.dot_general(w0[:, D1:], acc[...],
                                (((1,), (0,)), ((), ())),
                                preferred_element_type=jnp.float32)
          + b0_ref[...])
    y0_ref[...] = y0

    s1 = jnp.sum(y0, axis=1)
    s2 = jnp.sum(y0 * y0, axis=1)
    blk = jnp.concatenate(
        [s1[None, :], s2[None, :], jnp.zeros((6, C0), jnp.float32)], axis=0)

    @pl.when(i == 0)
    def _():
        stats_ref[...] = blk

    @pl.when(i != 0)
    def _():
        stats_ref[...] += blk


def _mid_body(y0_ref, sc_ref, sh_ref, w1_ref, b1_ref, y1_ref, stats_ref):
    h = jnp.maximum(y0_ref[...] * sc_ref[...] + sh_ref[...], 0.0)
    y1 = jax.lax.dot_general(w1_ref[...], h, (((1,), (0,)), ((), ())),
                             preferred_element_type=jnp.float32) + b1_ref[...]
    y1_ref[...] = y1
    s1 = jnp.sum(y1, axis=1)
    s2 = jnp.sum(y1 * y1, axis=1)
    blk = jnp.concatenate(
        [s1[None, :], s2[None, :], jnp.zeros((6, C1), jnp.float32)], axis=0)

    @pl.when(pl.program_id(0) == 0)
    def _():
        stats_ref[...] = blk

    @pl.when(pl.program_id(0) != 0)
    def _():
        stats_ref[...] += blk


def _final_body(y1_ref, sc_ref, sh_ref, out_ref):
    out_ref[...] = jnp.maximum(y1_ref[...] * sc_ref[...] + sh_ref[...], 0.0)


def _affine(stats, g, beta, n):
    mean = stats[0] / n
    var = stats[1] / n - mean * mean
    scale = g / jnp.sqrt(var + 1e-5)
    shift = beta - mean * scale
    return scale.reshape(-1, 1), shift.reshape(-1, 1)


@jax.jit
def kernel(xyz1, xyz2, piece_id1, piece_id2, points1, points2,
           conv_w0, conv_b0, bn_g0, bn_b0, conv_w1, conv_b1, bn_g1, bn_b1):
    x1 = xyz1[0]
    x2 = xyz2[0]
    p1 = piece_id1.reshape(1, N)
    p2 = piece_id2.reshape(1, S)
    pts1 = points1[0]
    pts2 = points2[0]
    b0 = conv_b0.reshape(C0, 1)
    b1 = conv_b1.reshape(C1, 1)

    grid1 = N // QB
    p2f = p2.reshape(-1)
    p1f = p1.reshape(-1)
    chunk_lo = p2f[::CB]
    chunk_hi = p2f[CB - 1::CB]
    tile_lo = p1f[::QB]
    tile_hi = p1f[QB - 1::QB]
    info = jnp.stack([chunk_lo, chunk_hi], axis=1).reshape(-1)
    info = jnp.concatenate(
        [info, jnp.stack([tile_lo, tile_hi], axis=1).reshape(-1)])

    y0, stats0 = pl.pallas_call(
        _knn_body,
        grid_spec=pltpu.PrefetchScalarGridSpec(
            num_scalar_prefetch=1,
            grid=(grid1,),
            in_specs=[
                pl.BlockSpec((3, QB), lambda i, info: (0, i)),
                pl.BlockSpec((1, QB), lambda i, info: (0, i)),
                pl.BlockSpec((3, S), lambda i, info: (0, 0)),
                pl.BlockSpec((1, S), lambda i, info: (0, 0)),
                pl.BlockSpec((D2, S), lambda i, info: (0, 0)),
                pl.BlockSpec((D1, QB), lambda i, info: (0, i)),
                pl.BlockSpec((C0, D1 + D2), lambda i, info: (0, 0)),
                pl.BlockSpec((C0, 1), lambda i, info: (0, 0)),
            ],
            out_specs=[
                pl.BlockSpec((C0, QB), lambda i, info: (0, i)),
                pl.BlockSpec((8, C0), lambda i, info: (0, 0)),
            ],
            scratch_shapes=[
                pltpu.VMEM((S, QB), jnp.float32),
                pltpu.VMEM((D2, QB), jnp.float32),
                pltpu.VMEM((8, QB), jnp.float32),
            ],
        ),
        out_shape=[
            jax.ShapeDtypeStruct((C0, N), jnp.float32),
            jax.ShapeDtypeStruct((8, C0), jnp.float32),
        ],
    )(info, x1, p1, x2, p2, pts2, pts1, conv_w0, b0)

    sc0, sh0 = _affine(stats0, bn_g0, bn_b0, N)

    grid2 = N // MB
    y1, stats1 = pl.pallas_call(
        _mid_body,
        grid=(grid2,),
        in_specs=[
            pl.BlockSpec((C0, MB), lambda i: (0, i)),
            pl.BlockSpec((C0, 1), lambda i: (0, 0)),
            pl.BlockSpec((C0, 1), lambda i: (0, 0)),
            pl.BlockSpec((C1, C0), lambda i: (0, 0)),
            pl.BlockSpec((C1, 1), lambda i: (0, 0)),
        ],
        out_specs=[
            pl.BlockSpec((C1, MB), lambda i: (0, i)),
            pl.BlockSpec((8, C1), lambda i: (0, 0)),
        ],
        out_shape=[
            jax.ShapeDtypeStruct((C1, N), jnp.float32),
            jax.ShapeDtypeStruct((8, C1), jnp.float32),
        ],
    )(y0, sc0, sh0, conv_w1, b1)

    sc1, sh1 = _affine(stats1, bn_g1, bn_b1, N)

    out = pl.pallas_call(
        _final_body,
        grid=(grid2,),
        in_specs=[
            pl.BlockSpec((C1, MB), lambda i: (0, i)),
            pl.BlockSpec((C1, 1), lambda i: (0, 0)),
            pl.BlockSpec((C1, 1), lambda i: (0, 0)),
        ],
        out_specs=pl.BlockSpec((C1, MB), lambda i: (0, i)),
        out_shape=jax.ShapeDtypeStruct((C1, N), jnp.float32),
    )(y1, sc1, sh1)

    return out[None]
